# baseline (speedup 1.0000x reference)
import jax
import jax.numpy as jnp
from jax.experimental import pallas as pl
from jax.experimental.pallas import tpu as pltpu

_LANES = 128


def _gather_body(x_ref, i_ref, o_ref):
    x = x_ref[...]
    idx = i_ref[...]
    T = x.shape[0]
    n_chunks = x.shape[1] // _LANES
    n_p = idx.shape[1] // _LANES

    lo = jnp.bitwise_and(idx, _LANES - 1)

    for rg in range(T // 8):
        rows = slice(rg * 8, rg * 8 + 8)
        for h in range(n_p):
            sl = slice(h * _LANES, (h + 1) * _LANES)
            lo_h = lo[rows, sl]
            idx_h = idx[rows, sl]
            gs = [jnp.take_along_axis(
                x[rows, c * _LANES:(c + 1) * _LANES], lo_h, axis=1)
                for c in range(n_chunks)]
            bit = _LANES
            while len(gs) > 1:
                take_lo = (idx_h & bit) == 0
                gs = [jnp.where(take_lo, gs[2 * i], gs[2 * i + 1])
                      for i in range(len(gs) // 2)]
                bit *= 2
            o_ref[rows, sl] = gs[0]


def _gather_2d(x2d, idx2d, tile_r=4096):
    R, M = x2d.shape
    _, P = idx2d.shape
    assert M % _LANES == 0 and P % _LANES == 0 and R % tile_r == 0

    grid = (R // tile_r,)
    return pl.pallas_call(
        _gather_body,
        out_shape=jax.ShapeDtypeStruct((R, P), x2d.dtype),
        grid=grid,
        in_specs=[
            pl.BlockSpec((tile_r, M), lambda i: (i, 0)),
            pl.BlockSpec((tile_r, P), lambda i: (i, 0)),
        ],
        out_specs=pl.BlockSpec((tile_r, P), lambda i: (i, 0)),
        compiler_params=pltpu.CompilerParams(
            dimension_semantics=("parallel",),
            vmem_limit_bytes=60 * 1024 * 1024,
        ),
    )(x2d, idx2d)


def kernel(inp, index):
    batch_shape = index.shape[:-1]
    M = inp.shape[-1]
    P = index.shape[-1]
    x2 = inp.reshape(-1, M)
    i2 = index.reshape(-1, P).astype(jnp.int32)
    out2 = _gather_2d(x2, i2)
    return out2.reshape(*batch_shape, P).astype(inp.dtype)


# speedup vs baseline: 1.0619x; 1.0619x over previous
import jax
import jax.numpy as jnp
from jax.experimental import pallas as pl
from jax.experimental.pallas import tpu as pltpu

_LANES = 128


def _gather_body(x_ref, i_ref, o_ref):
    x = x_ref[...]
    idx = i_ref[...]
    T = x.shape[0]
    n_chunks = x.shape[1] // _LANES
    n_p = idx.shape[1] // _LANES

    lo = jnp.bitwise_and(idx, _LANES - 1)

    for rg in range(T // 8):
        rows = slice(rg * 8, rg * 8 + 8)
        for h in range(n_p):
            sl = slice(h * _LANES, (h + 1) * _LANES)
            lo_h = lo[rows, sl]
            idx_h = idx[rows, sl]
            gs = [jnp.take_along_axis(
                x[rows, c * _LANES:(c + 1) * _LANES], lo_h, axis=1)
                for c in range(n_chunks)]
            bit = _LANES
            while len(gs) > 1:
                take_lo = (idx_h & bit) == 0
                gs = [jnp.where(take_lo, gs[2 * i], gs[2 * i + 1])
                      for i in range(len(gs) // 2)]
                bit *= 2
            o_ref[rows, sl] = gs[0]


def _gather_2d(x2d, idx2d, tile_r=2048):
    R, M = x2d.shape
    _, P = idx2d.shape
    assert M % _LANES == 0 and P % _LANES == 0 and R % tile_r == 0

    grid = (R // tile_r,)
    return pl.pallas_call(
        _gather_body,
        out_shape=jax.ShapeDtypeStruct((R, P), x2d.dtype),
        grid=grid,
        in_specs=[
            pl.BlockSpec((tile_r, M), lambda i: (i, 0)),
            pl.BlockSpec((tile_r, P), lambda i: (i, 0)),
        ],
        out_specs=pl.BlockSpec((tile_r, P), lambda i: (i, 0)),
        compiler_params=pltpu.CompilerParams(
            dimension_semantics=("parallel",),
            vmem_limit_bytes=60 * 1024 * 1024,
        ),
    )(x2d, idx2d)


def kernel(inp, index):
    batch_shape = index.shape[:-1]
    M = inp.shape[-1]
    P = index.shape[-1]
    x2 = inp.reshape(-1, M)
    i2 = index.reshape(-1, P).astype(jnp.int32)
    out2 = _gather_2d(x2, i2)
    return out2.reshape(*batch_shape, P).astype(inp.dtype)
